# Initial kernel scaffold; baseline (speedup 1.0000x reference)
import jax
import jax.numpy as jnp
from jax.experimental import pallas as pl


def kernel(in_features, neighbors_index, neighbors_row_splits):
    raise NotImplementedError("write your pallas kernel here")


# speedup vs baseline: 16.6243x; 16.6243x over previous
import functools

import jax
import jax.numpy as jnp
from jax import lax
from jax.experimental import pallas as pl
from jax.experimental.pallas import tpu as pltpu
from jax.experimental.pallas import tpu_sc as plsc


def kernel(in_features, neighbors_index, neighbors_row_splits):
    N, C = in_features.shape
    E = neighbors_index.shape[0]
    M = neighbors_row_splits.shape[0] - 1

    info = plsc.get_sparse_core_info()
    NCORES, NSUB = info.num_cores, info.num_subcores
    NW = NCORES * NSUB
    MPW = (M + NW - 1) // NW
    G = 16
    LAN = 16
    KC = C // LAN

    idx32 = jnp.pad(neighbors_index.astype(jnp.int32), (0, 32))
    rs32 = jnp.pad(neighbors_row_splits.astype(jnp.int32), (0, 64))
    RSBUF = ((MPW + 1 + 7 + 15 + 7) // 8) * 8

    mesh = plsc.VectorSubcoreMesh(core_axis_name="c", subcore_axis_name="s")

    @functools.partial(
        pl.kernel,
        mesh=mesh,
        out_type=jax.ShapeDtypeStruct((M, C), jnp.float32),
        scratch_types=[
            pltpu.VMEM((RSBUF,), jnp.int32),
            pltpu.VMEM((G,), jnp.int32),
            pltpu.VMEM((G, C), jnp.float32),
            pltpu.VMEM((C,), jnp.float32),
            pltpu.SemaphoreType.DMA,
        ],
    )
    def pool(feat_hbm, idx_hbm, rs_hbm, out_hbm, rs_v, idx_v, g_v, row_v, sem):
        wid = lax.axis_index("s") * NCORES + lax.axis_index("c")
        m0 = wid * MPW
        mcount = jnp.minimum(MPW, M - m0)
        rs_astart = (m0 // 8) * 8
        rs_off = m0 - rs_astart
        pltpu.sync_copy(rs_hbm.at[pl.ds(rs_astart, RSBUF)], rs_v)

        def seg_body(j, carry):
            rsv = rs_v[pl.ds(rs_off + j, 16)]
            base = rsv[0]
            nxt = rsv[1]
            cnt = nxt - base
            astart = (base // 8) * 8
            pre = base - astart
            total = pre + cnt
            nch = jnp.where(cnt > 0, (total + G - 1) // G, 0)

            def chunk_body(t, accs):
                pltpu.sync_copy(idx_hbm.at[pl.ds(astart + t * G, G)], idx_v)
                pltpu.async_copy(feat_hbm.at[idx_v], g_v, sem).wait()
                lo = jnp.maximum(0, pre - t * G)
                hi = jnp.minimum(G, total - t * G)

                def edge_body(e, accs2):
                    return [accs2[k] + g_v[e, pl.ds(k * LAN, LAN)]
                            for k in range(KC)]

                return lax.fori_loop(lo, hi, edge_body, accs)

            zero = jnp.zeros((LAN,), jnp.float32)
            accs = lax.fori_loop(0, nch, chunk_body, [zero] * KC)
            cntv = jnp.full((LAN,), cnt.astype(jnp.float32))
            recip = 1.0 / jnp.maximum(cntv, 1.0)
            for k in range(KC):
                row_v[pl.ds(k * LAN, LAN)] = accs[k] * recip
            pltpu.sync_copy(row_v, out_hbm.at[m0 + j])
            return carry

        lax.fori_loop(0, mcount, seg_body, 0)

    return pool(in_features, idx32, rs32)


# speedup vs baseline: 42.2630x; 2.5422x over previous
import functools

import jax
import jax.numpy as jnp
from jax import lax
from jax.experimental import pallas as pl
from jax.experimental.pallas import tpu as pltpu
from jax.experimental.pallas import tpu_sc as plsc


def kernel(in_features, neighbors_index, neighbors_row_splits):
    N, C = in_features.shape
    E = neighbors_index.shape[0]
    M = neighbors_row_splits.shape[0] - 1

    info = plsc.get_sparse_core_info()
    NCORES, NSUB = info.num_cores, info.num_subcores
    NW = NCORES * NSUB
    MPW = -(-((M + NW - 1) // NW) // 8) * 8
    MLAST = M - (NW - 1) * MPW
    G = 16
    BLK = 2048
    LAN = 16
    KC = C // LAN

    RSBUF = ((MPW + 2 + 7 + 15 + 7) // 8) * 8
    idx32 = jnp.pad(neighbors_index.astype(jnp.int32), (0, BLK + G + 8))
    rs32 = jnp.pad(neighbors_row_splits.astype(jnp.int32), (0, RSBUF),
                   constant_values=E)

    mesh = plsc.VectorSubcoreMesh(core_axis_name="c", subcore_axis_name="s")

    @functools.partial(
        pl.kernel,
        mesh=mesh,
        out_type=jax.ShapeDtypeStruct((M, C), jnp.float32),
        scratch_types=[
            pltpu.VMEM((RSBUF,), jnp.int32),
            pltpu.VMEM((BLK,), jnp.int32),
            pltpu.VMEM((2, G, C), jnp.float32),
            pltpu.VMEM((MPW, C), jnp.float32),
            pltpu.SemaphoreType.DMA((2,)),
        ],
    )
    def pool(feat_hbm, idx_hbm, rs_hbm, out_hbm,
             rs_v, blk_v, g_v, out_v, gsem):
        wid = lax.axis_index("s") * NCORES + lax.axis_index("c")
        m0 = pl.multiple_of(wid * MPW, 8)
        mcount = jnp.minimum(MPW, M - m0)
        rs_astart = pl.multiple_of((m0 // 8) * 8, 8)
        rs_off = m0 - rs_astart
        pltpu.sync_copy(rs_hbm.at[pl.ds(rs_astart, RSBUF)], rs_v)

        def rs_at(i):
            return rs_v[pl.ds(rs_off + i, LAN)][0]

        e0 = rs_at(0)
        eN = rs_at(mcount)
        gstart = pl.multiple_of((e0 // 8) * 8, 8)
        nchunks = (eN - gstart + G - 1) // G
        nevents = nchunks + mcount

        zero = jnp.zeros((LAN,), jnp.float32)
        zeros_kc = (zero,) * KC

        @pl.when(nchunks > 0)
        def _():
            pltpu.sync_copy(idx_hbm.at[pl.ds(gstart, BLK)], blk_v)
            pltpu.async_copy(feat_hbm.at[blk_v[pl.ds(0, G)]],
                             g_v.at[0], gsem.at[0])

            @pl.when(nchunks > 1)
            def _():
                pltpu.async_copy(feat_hbm.at[blk_v[pl.ds(G, G)]],
                                 g_v.at[1], gsem.at[1])

            pltpu.make_async_copy(feat_hbm.at[pl.ds(0, G)], g_v.at[0],
                                  gsem.at[0]).wait()

        def event(_, st):
            c, cur, m, seg_start, seg_end, p, bstart, accs = st
            cs = gstart + c * G
            cend = jnp.minimum(cs + G, eN)

            take = jnp.maximum(jnp.minimum(seg_end, cend) - cur, 0)
            lo = cur - cs

            def edge_body(e, a):
                return tuple(a[k] + g_v[p, e, pl.ds(k * LAN, LAN)]
                             for k in range(KC))

            accs = lax.fori_loop(lo, lo + take, edge_body, accs)
            cur = cur + take

            hit = jnp.logical_and(cur >= seg_end, m < mcount)
            adv = jnp.logical_and(jnp.logical_not(hit),
                                  jnp.logical_and(cur >= cend,
                                                  c + 1 < nchunks))

            @pl.when(hit)
            def _():
                cnt = seg_end - seg_start
                cntv = jnp.full((LAN,), cnt.astype(jnp.float32))
                recip = 1.0 / jnp.maximum(cntv, 1.0)
                for k in range(KC):
                    out_v[m, pl.ds(k * LAN, LAN)] = accs[k] * recip

            issue = jnp.logical_and(adv, c + 2 < nchunks)
            naddr = pl.multiple_of(gstart + (c + 2) * G, 8)
            restage = jnp.logical_and(issue, naddr + G > bstart + BLK)
            nbstart = jnp.where(restage, naddr, bstart)

            @pl.when(adv)
            def _():
                pltpu.make_async_copy(feat_hbm.at[pl.ds(0, G)],
                                      g_v.at[1 - p], gsem.at[1 - p]).wait()

                @pl.when(issue)
                def _():
                    @pl.when(restage)
                    def _():
                        pltpu.sync_copy(idx_hbm.at[pl.ds(naddr, BLK)], blk_v)

                    idxv = blk_v[pl.ds(naddr - nbstart, G)]
                    pltpu.async_copy(feat_hbm.at[idxv], g_v.at[p], gsem.at[p])

            nm = jnp.where(hit, m + 1, m)
            seg_start = jnp.where(hit, seg_end, seg_start)
            seg_end = jnp.where(hit, rs_at(nm + 1), seg_end)
            accs = tuple(jnp.where(hit, zero, a) for a in accs)
            c = jnp.where(adv, c + 1, c)
            p = jnp.where(adv, 1 - p, p)
            return (c, cur, nm, seg_start, seg_end, p, nbstart, accs)

        st0 = (jnp.int32(0), e0, jnp.int32(0), e0, rs_at(1),
               jnp.int32(0), gstart, zeros_kc)
        lax.fori_loop(0, nevents, event, st0)

        @pl.when(wid < NW - 1)
        def _():
            pltpu.sync_copy(out_v, out_hbm.at[pl.ds(m0, MPW)])

        @pl.when(wid == NW - 1)
        def _():
            pltpu.sync_copy(out_v.at[pl.ds(0, MLAST)],
                            out_hbm.at[pl.ds(m0, MLAST)])

    return pool(in_features, idx32, rs32)


# speedup vs baseline: 102.9538x; 2.4360x over previous
import functools

import jax
import jax.numpy as jnp
from jax import lax
from jax.experimental import pallas as pl
from jax.experimental.pallas import tpu as pltpu
from jax.experimental.pallas import tpu_sc as plsc


def kernel(in_features, neighbors_index, neighbors_row_splits):
    N, C = in_features.shape
    E = neighbors_index.shape[0]
    M = neighbors_row_splits.shape[0] - 1

    info = plsc.get_sparse_core_info()
    NCORES, NSUB = info.num_cores, info.num_subcores
    NW = NCORES * NSUB
    MPW = -(-((M + NW - 1) // NW) // 8) * 8
    MLAST = M - (NW - 1) * MPW
    G = 32
    R = 4
    BLK = 2048
    LAN = 16
    KC = C // LAN

    RSBUF = ((MPW + 2 + 7 + 15 + 7) // 8) * 8
    idx32 = jnp.pad(neighbors_index.astype(jnp.int32), (0, BLK + G + 8))
    rs32 = jnp.pad(neighbors_row_splits.astype(jnp.int32), (0, RSBUF),
                   constant_values=E)

    mesh = plsc.VectorSubcoreMesh(core_axis_name="c", subcore_axis_name="s")

    @functools.partial(
        pl.kernel,
        mesh=mesh,
        out_type=jax.ShapeDtypeStruct((M, C), jnp.float32),
        scratch_types=[
            pltpu.VMEM((RSBUF,), jnp.int32),
            pltpu.VMEM((BLK,), jnp.int32),
            pltpu.VMEM((R, G, C), jnp.float32),
            pltpu.VMEM((MPW, C), jnp.float32),
            pltpu.SemaphoreType.DMA((R,)),
        ],
    )
    def pool(feat_hbm, idx_hbm, rs_hbm, out_hbm,
             rs_v, blk_v, g_v, out_v, gsem):
        wid = lax.axis_index("s") * NCORES + lax.axis_index("c")
        m0 = pl.multiple_of(wid * MPW, 8)
        mcount = jnp.minimum(MPW, M - m0)
        rs_astart = pl.multiple_of((m0 // 8) * 8, 8)
        rs_off = m0 - rs_astart
        pltpu.sync_copy(rs_hbm.at[pl.ds(rs_astart, RSBUF)], rs_v)

        def rs_at(i):
            return rs_v[pl.ds(rs_off + i, LAN)][0]

        e0 = rs_at(0)
        eN = rs_at(mcount)
        gstart = pl.multiple_of((e0 // 8) * 8, 8)
        nchunks = (eN - gstart + G - 1) // G
        nevents = nchunks + mcount

        zero = jnp.zeros((LAN,), jnp.float32)
        zeros_kc = (zero,) * KC

        def issue_gather(off, slot_ds):
            for h in range(G // LAN):
                idxv = blk_v[pl.ds(off + h * LAN, LAN)]
                pltpu.async_copy(feat_hbm.at[idxv],
                                 g_v.at[slot_ds, pl.ds(h * LAN, LAN)],
                                 gsem.at[slot_ds])

        @pl.when(nchunks > 0)
        def _():
            pltpu.sync_copy(idx_hbm.at[pl.ds(gstart, BLK)], blk_v)
            for i in range(R - 1):
                @pl.when(nchunks > i)
                def _(i=i):
                    issue_gather(i * G, i)
            pltpu.make_async_copy(feat_hbm.at[pl.ds(0, G)], g_v.at[0],
                                  gsem.at[0]).wait()

        def event(_, st):
            c, cur, m, seg_start, seg_end, bstart, accs = st
            p = lax.rem(c, R)
            cs = gstart + c * G
            cend = jnp.minimum(cs + G, eN)

            take = jnp.maximum(jnp.minimum(seg_end, cend) - cur, 0)
            lo = cur - cs

            def edge_body(e, a):
                return tuple(a[k] + g_v[p, e, pl.ds(k * LAN, LAN)]
                             for k in range(KC))

            accs = lax.fori_loop(lo, lo + take, edge_body, accs)
            cur = cur + take

            hit = jnp.logical_and(cur >= seg_end, m < mcount)
            adv = jnp.logical_and(jnp.logical_not(hit),
                                  jnp.logical_and(cur >= cend,
                                                  c + 1 < nchunks))

            @pl.when(hit)
            def _():
                cnt = seg_end - seg_start
                cntv = jnp.full((LAN,), cnt.astype(jnp.float32))
                recip = 1.0 / jnp.maximum(cntv, 1.0)
                for k in range(KC):
                    out_v[m, pl.ds(k * LAN, LAN)] = accs[k] * recip

            issue = jnp.logical_and(adv, c + R - 1 < nchunks)
            naddr = pl.multiple_of(gstart + (c + R - 1) * G, 8)
            restage = jnp.logical_and(issue, naddr + G > bstart + BLK)
            nbstart = jnp.where(restage, naddr, bstart)

            @pl.when(adv)
            def _():
                ws = lax.rem(c + 1, R)
                pltpu.make_async_copy(feat_hbm.at[pl.ds(0, G)],
                                      g_v.at[ws], gsem.at[ws]).wait()

                @pl.when(issue)
                def _():
                    @pl.when(restage)
                    def _():
                        pltpu.sync_copy(idx_hbm.at[pl.ds(naddr, BLK)], blk_v)

                    issue_gather(naddr - nbstart, lax.rem(c + R - 1, R))

            nm = jnp.where(hit, m + 1, m)
            seg_start = jnp.where(hit, seg_end, seg_start)
            seg_end = jnp.where(hit, rs_at(nm + 1), seg_end)
            accs = tuple(jnp.where(hit, zero, a) for a in accs)
            c = jnp.where(adv, c + 1, c)
            return (c, cur, nm, seg_start, seg_end, nbstart, accs)

        st0 = (jnp.int32(0), e0, jnp.int32(0), e0, rs_at(1),
               gstart, zeros_kc)
        lax.fori_loop(0, nevents, event, st0)

        @pl.when(wid < NW - 1)
        def _():
            pltpu.sync_copy(out_v, out_hbm.at[pl.ds(m0, MPW)])

        @pl.when(wid == NW - 1)
        def _():
            pltpu.sync_copy(out_v.at[pl.ds(0, MLAST)],
                            out_hbm.at[pl.ds(m0, MLAST)])

    return pool(in_features, idx32, rs32)


# speedup vs baseline: 112.1778x; 1.0896x over previous
import functools

import jax
import jax.numpy as jnp
from jax import lax
from jax.experimental import pallas as pl
from jax.experimental.pallas import tpu as pltpu
from jax.experimental.pallas import tpu_sc as plsc


def kernel(in_features, neighbors_index, neighbors_row_splits):
    N, C = in_features.shape
    E = neighbors_index.shape[0]
    M = neighbors_row_splits.shape[0] - 1

    info = plsc.get_sparse_core_info()
    NCORES, NSUB = info.num_cores, info.num_subcores
    NW = NCORES * NSUB
    MPW = -(-((M + NW - 1) // NW) // 8) * 8
    MLAST = M - (NW - 1) * MPW
    G = 32
    R = 5
    BLK = 2048
    LAN = 16
    KC = C // LAN

    RSBUF = ((MPW + 2 + 7 + 15 + 7) // 8) * 8
    idx32 = jnp.pad(neighbors_index.astype(jnp.int32), (0, BLK + G + 8))
    rs32 = jnp.pad(neighbors_row_splits.astype(jnp.int32), (0, RSBUF),
                   constant_values=E)

    mesh = plsc.VectorSubcoreMesh(core_axis_name="c", subcore_axis_name="s")

    @functools.partial(
        pl.kernel,
        mesh=mesh,
        out_type=jax.ShapeDtypeStruct((M, C), jnp.float32),
        scratch_types=[
            pltpu.VMEM((RSBUF,), jnp.int32),
            pltpu.VMEM((BLK,), jnp.int32),
            pltpu.VMEM((R, G, C), jnp.float32),
            pltpu.VMEM((MPW, C), jnp.float32),
            pltpu.SemaphoreType.DMA((R,)),
        ],
    )
    def pool(feat_hbm, idx_hbm, rs_hbm, out_hbm,
             rs_v, blk_v, g_v, out_v, gsem):
        wid = lax.axis_index("s") * NCORES + lax.axis_index("c")
        m0 = pl.multiple_of(wid * MPW, 8)
        mcount = jnp.minimum(MPW, M - m0)
        rs_astart = pl.multiple_of((m0 // 8) * 8, 8)
        rs_off = m0 - rs_astart
        pltpu.sync_copy(rs_hbm.at[pl.ds(rs_astart, RSBUF)], rs_v)

        def rs_at(i):
            return rs_v[pl.ds(rs_off + i, LAN)][0]

        e0 = rs_at(0)
        eN = rs_at(mcount)
        gstart = pl.multiple_of((e0 // 8) * 8, 8)
        nchunks = (eN - gstart + G - 1) // G
        nevents = nchunks + mcount

        zero = jnp.zeros((LAN,), jnp.float32)
        zeros_kc = (zero,) * KC

        def issue_gather(off, slot_ds):
            for h in range(G // LAN):
                idxv = blk_v[pl.ds(off + h * LAN, LAN)]
                pltpu.async_copy(feat_hbm.at[idxv],
                                 g_v.at[slot_ds, pl.ds(h * LAN, LAN)],
                                 gsem.at[slot_ds])

        @pl.when(nchunks > 0)
        def _():
            pltpu.sync_copy(idx_hbm.at[pl.ds(gstart, BLK)], blk_v)
            for i in range(R - 1):
                @pl.when(nchunks > i)
                def _(i=i):
                    issue_gather(i * G, i)
            pltpu.make_async_copy(feat_hbm.at[pl.ds(0, G)], g_v.at[0],
                                  gsem.at[0]).wait()

        def event(_, st):
            c, cur, m, seg_start, seg_end, bstart, accs = st
            p = lax.rem(c, R)
            cs = gstart + c * G
            cend = jnp.minimum(cs + G, eN)

            take = jnp.maximum(jnp.minimum(seg_end, cend) - cur, 0)
            lo = cur - cs

            def edge_body(e, a):
                return tuple(a[k] + g_v[p, e, pl.ds(k * LAN, LAN)]
                             for k in range(KC))

            accs = lax.fori_loop(lo, lo + take, edge_body, accs)
            cur = cur + take

            hit = jnp.logical_and(cur >= seg_end, m < mcount)
            adv = jnp.logical_and(jnp.logical_not(hit),
                                  jnp.logical_and(cur >= cend,
                                                  c + 1 < nchunks))

            @pl.when(hit)
            def _():
                cnt = seg_end - seg_start
                cntv = jnp.full((LAN,), cnt.astype(jnp.float32))
                recip = 1.0 / jnp.maximum(cntv, 1.0)
                for k in range(KC):
                    out_v[m, pl.ds(k * LAN, LAN)] = accs[k] * recip

            issue = jnp.logical_and(adv, c + R - 1 < nchunks)
            naddr = pl.multiple_of(gstart + (c + R - 1) * G, 8)
            restage = jnp.logical_and(issue, naddr + G > bstart + BLK)
            nbstart = jnp.where(restage, naddr, bstart)

            @pl.when(adv)
            def _():
                ws = lax.rem(c + 1, R)
                pltpu.make_async_copy(feat_hbm.at[pl.ds(0, G)],
                                      g_v.at[ws], gsem.at[ws]).wait()

                @pl.when(issue)
                def _():
                    @pl.when(restage)
                    def _():
                        pltpu.sync_copy(idx_hbm.at[pl.ds(naddr, BLK)], blk_v)

                    issue_gather(naddr - nbstart, lax.rem(c + R - 1, R))

            nm = jnp.where(hit, m + 1, m)
            seg_start = jnp.where(hit, seg_end, seg_start)
            seg_end = jnp.where(hit, rs_at(nm + 1), seg_end)
            accs = tuple(jnp.where(hit, zero, a) for a in accs)
            c = jnp.where(adv, c + 1, c)
            return (c, cur, nm, seg_start, seg_end, nbstart, accs)

        st0 = (jnp.int32(0), e0, jnp.int32(0), e0, rs_at(1),
               gstart, zeros_kc)
        lax.fori_loop(0, nevents, event, st0)

        @pl.when(wid < NW - 1)
        def _():
            pltpu.sync_copy(out_v, out_hbm.at[pl.ds(m0, MPW)])

        @pl.when(wid == NW - 1)
        def _():
            pltpu.sync_copy(out_v.at[pl.ds(0, MLAST)],
                            out_hbm.at[pl.ds(m0, MLAST)])

    return pool(in_features, idx32, rs32)


# speedup vs baseline: 113.2257x; 1.0093x over previous
import functools

import jax
import jax.numpy as jnp
from jax import lax
from jax.experimental import pallas as pl
from jax.experimental.pallas import tpu as pltpu
from jax.experimental.pallas import tpu_sc as plsc


def kernel(in_features, neighbors_index, neighbors_row_splits):
    N, C = in_features.shape
    E = neighbors_index.shape[0]
    M = neighbors_row_splits.shape[0] - 1

    info = plsc.get_sparse_core_info()
    NCORES, NSUB = info.num_cores, info.num_subcores
    NW = NCORES * NSUB
    MPW = -(-((M + NW - 1) // NW) // 8) * 8
    MLAST = M - (NW - 1) * MPW
    G = 32
    R = 5
    BLK = 2048
    LAN = 16
    KC = C // LAN

    RSBUF = ((MPW + 2 + 7 + 15 + 7) // 8) * 8
    idx32 = jnp.pad(neighbors_index.astype(jnp.int32), (0, BLK + G + 8))
    rs32 = jnp.pad(neighbors_row_splits.astype(jnp.int32), (0, RSBUF),
                   constant_values=E)

    mesh = plsc.VectorSubcoreMesh(core_axis_name="c", subcore_axis_name="s")

    @functools.partial(
        pl.kernel,
        mesh=mesh,
        out_type=jax.ShapeDtypeStruct((M, C), jnp.float32),
        scratch_types=[
            pltpu.VMEM((RSBUF,), jnp.int32),
            pltpu.VMEM((BLK,), jnp.int32),
            pltpu.VMEM((R, G, C), jnp.float32),
            pltpu.VMEM((MPW, C), jnp.float32),
            pltpu.SemaphoreType.DMA((R,)),
        ],
    )
    def pool(feat_hbm, idx_hbm, rs_hbm, out_hbm,
             rs_v, blk_v, g_v, out_v, gsem):
        wid = lax.axis_index("s") * NCORES + lax.axis_index("c")
        m0 = pl.multiple_of(wid * MPW, 8)
        mcount = jnp.minimum(MPW, M - m0)
        rs_astart = pl.multiple_of((m0 // 8) * 8, 8)
        rs_off = m0 - rs_astart
        pltpu.sync_copy(rs_hbm.at[pl.ds(rs_astart, RSBUF)], rs_v)

        def rs_at(i):
            return rs_v[pl.ds(rs_off + i, LAN)][0]

        e0 = rs_at(0)
        eN = rs_at(mcount)
        gstart = pl.multiple_of((e0 // 8) * 8, 8)
        nchunks = (eN - gstart + G - 1) // G
        nevents = nchunks + mcount

        zero = jnp.zeros((LAN,), jnp.float32)
        zeros_kc = (zero,) * KC

        def issue_gather(off, slot_ds):
            for h in range(G // LAN):
                idxv = blk_v[pl.ds(off + h * LAN, LAN)]
                pltpu.async_copy(feat_hbm.at[idxv],
                                 g_v.at[slot_ds, pl.ds(h * LAN, LAN)],
                                 gsem.at[slot_ds])

        @pl.when(nchunks > 0)
        def _():
            pltpu.sync_copy(idx_hbm.at[pl.ds(gstart, BLK)], blk_v)
            for i in range(R - 1):
                @pl.when(nchunks > i)
                def _(i=i):
                    issue_gather(i * G, i)
            pltpu.make_async_copy(feat_hbm.at[pl.ds(0, G)], g_v.at[0],
                                  gsem.at[0]).wait()

        def event(_, st):
            c, cur, m, seg_start, seg_end, bstart, accs = st
            p = lax.rem(c, R)
            cs = gstart + c * G
            cend = jnp.minimum(cs + G, eN)

            take = jnp.maximum(jnp.minimum(seg_end, cend) - cur, 0)
            lo = cur - cs

            def edge_body(e, a):
                return tuple(a[k] + g_v[p, e, pl.ds(k * LAN, LAN)]
                             for k in range(KC))

            accs = lax.fori_loop(lo, lo + take, edge_body, accs)
            cur = cur + take

            hit = jnp.logical_and(cur >= seg_end, m < mcount)
            adv = jnp.logical_and(jnp.logical_not(hit),
                                  jnp.logical_and(cur >= cend,
                                                  c + 1 < nchunks))

            @pl.when(hit)
            def _():
                cnt = seg_end - seg_start
                cntv = jnp.full((LAN,), cnt.astype(jnp.float32))
                recip = 1.0 / jnp.maximum(cntv, 1.0)
                for k in range(KC):
                    out_v[m, pl.ds(k * LAN, LAN)] = accs[k] * recip

            issue = jnp.logical_and(adv, c + R - 1 < nchunks)
            naddr = pl.multiple_of(gstart + (c + R - 1) * G, 8)
            restage = jnp.logical_and(issue, naddr + G > bstart + BLK)
            nbstart = jnp.where(restage, naddr, bstart)

            @pl.when(adv)
            def _():
                @pl.when(issue)
                def _():
                    @pl.when(restage)
                    def _():
                        pltpu.sync_copy(idx_hbm.at[pl.ds(naddr, BLK)], blk_v)

                    issue_gather(naddr - nbstart, lax.rem(c + R - 1, R))

                ws = lax.rem(c + 1, R)
                pltpu.make_async_copy(feat_hbm.at[pl.ds(0, G)],
                                      g_v.at[ws], gsem.at[ws]).wait()

            nm = jnp.where(hit, m + 1, m)
            seg_start = jnp.where(hit, seg_end, seg_start)
            seg_end = jnp.where(hit, rs_at(nm + 1), seg_end)
            accs = tuple(jnp.where(hit, zero, a) for a in accs)
            c = jnp.where(adv, c + 1, c)
            return (c, cur, nm, seg_start, seg_end, nbstart, accs)

        st0 = (jnp.int32(0), e0, jnp.int32(0), e0, rs_at(1),
               gstart, zeros_kc)
        lax.fori_loop(0, nevents, event, st0)

        @pl.when(wid < NW - 1)
        def _():
            pltpu.sync_copy(out_v, out_hbm.at[pl.ds(m0, MPW)])

        @pl.when(wid == NW - 1)
        def _():
            pltpu.sync_copy(out_v.at[pl.ds(0, MLAST)],
                            out_hbm.at[pl.ds(m0, MLAST)])

    return pool(in_features, idx32, rs32)


# speedup vs baseline: 113.7720x; 1.0048x over previous
import functools

import jax
import jax.numpy as jnp
from jax import lax
from jax.experimental import pallas as pl
from jax.experimental.pallas import tpu as pltpu
from jax.experimental.pallas import tpu_sc as plsc


def kernel(in_features, neighbors_index, neighbors_row_splits):
    N, C = in_features.shape
    E = neighbors_index.shape[0]
    M = neighbors_row_splits.shape[0] - 1

    info = plsc.get_sparse_core_info()
    NCORES, NSUB = info.num_cores, info.num_subcores
    NW = NCORES * NSUB
    MPW = -(-((M + NW - 1) // NW) // 8) * 8
    MLAST = M - (NW - 1) * MPW
    G = 32
    R = 8
    W = 4
    BLK = 2048
    LAN = 16
    KC = C // LAN

    RSBUF = ((MPW + 2 + 7 + 15 + 7) // 8) * 8
    idx32 = jnp.pad(neighbors_index.astype(jnp.int32), (0, BLK + G + 8))
    rs32 = jnp.pad(neighbors_row_splits.astype(jnp.int32), (0, RSBUF),
                   constant_values=E)

    mesh = plsc.VectorSubcoreMesh(core_axis_name="c", subcore_axis_name="s")

    @functools.partial(
        pl.kernel,
        mesh=mesh,
        out_type=jax.ShapeDtypeStruct((M, C), jnp.float32),
        scratch_types=[
            pltpu.VMEM((RSBUF,), jnp.int32),
            pltpu.VMEM((BLK,), jnp.int32),
            pltpu.VMEM((R, G, C), jnp.float32),
            pltpu.VMEM((W, 1, C), jnp.float32),
            pltpu.SemaphoreType.DMA((R,)),
            pltpu.SemaphoreType.DMA((W,)),
        ],
    )
    def pool(feat_hbm, idx_hbm, rs_hbm, out_hbm,
             rs_v, blk_v, g_v, row_v, gsem, wsem):
        wid = lax.axis_index("s") * NCORES + lax.axis_index("c")
        m0 = pl.multiple_of(wid * MPW, 8)
        mcount = jnp.minimum(MPW, M - m0)
        rs_astart = pl.multiple_of((m0 // 8) * 8, 8)
        rs_off = m0 - rs_astart
        pltpu.sync_copy(rs_hbm.at[pl.ds(rs_astart, RSBUF)], rs_v)

        def rs_at(i):
            return rs_v[pl.ds(rs_off + i, LAN)][0]

        e0 = rs_at(0)
        eN = rs_at(mcount)
        gstart = pl.multiple_of((e0 // 8) * 8, 8)
        nchunks = (eN - gstart + G - 1) // G
        nevents = nchunks + mcount

        zero = jnp.zeros((LAN,), jnp.float32)
        zeros_kc = (zero,) * KC

        def issue_gather(off, slot_ds):
            for h in range(G // LAN):
                idxv = blk_v[pl.ds(off + h * LAN, LAN)]
                pltpu.async_copy(feat_hbm.at[idxv],
                                 g_v.at[slot_ds, pl.ds(h * LAN, LAN)],
                                 gsem.at[slot_ds])

        @pl.when(nchunks > 0)
        def _():
            pltpu.sync_copy(idx_hbm.at[pl.ds(gstart, BLK)], blk_v)
            for i in range(R - 1):
                @pl.when(nchunks > i)
                def _(i=i):
                    issue_gather(i * G, i)
            pltpu.make_async_copy(feat_hbm.at[pl.ds(0, G)], g_v.at[0],
                                  gsem.at[0]).wait()

        def event(_, st):
            c, cur, m, seg_start, seg_end, bstart, accs = st
            p = lax.rem(c, R)
            cs = gstart + c * G
            cend = jnp.minimum(cs + G, eN)

            take = jnp.maximum(jnp.minimum(seg_end, cend) - cur, 0)
            lo = cur - cs

            def edge_body(e, a):
                return tuple(a[k] + g_v[p, e, pl.ds(k * LAN, LAN)]
                             for k in range(KC))

            accs = lax.fori_loop(lo, lo + take, edge_body, accs)
            cur = cur + take

            hit = jnp.logical_and(cur >= seg_end, m < mcount)
            adv = jnp.logical_and(jnp.logical_not(hit),
                                  jnp.logical_and(cur >= cend,
                                                  c + 1 < nchunks))

            @pl.when(hit)
            def _():
                ws2 = lax.rem(m, W)

                @pl.when(m >= W)
                def _():
                    pltpu.make_async_copy(feat_hbm.at[0], row_v.at[ws2, 0],
                                          wsem.at[ws2]).wait()

                cnt = seg_end - seg_start
                cntv = jnp.full((LAN,), cnt.astype(jnp.float32))
                recip = 1.0 / jnp.maximum(cntv, 1.0)
                for k in range(KC):
                    row_v[ws2, 0, pl.ds(k * LAN, LAN)] = accs[k] * recip
                pltpu.async_copy(row_v.at[ws2, 0], out_hbm.at[m0 + m],
                                 wsem.at[ws2])

            issue = jnp.logical_and(adv, c + R - 1 < nchunks)
            naddr = pl.multiple_of(gstart + (c + R - 1) * G, 8)
            restage = jnp.logical_and(issue, naddr + G > bstart + BLK)
            nbstart = jnp.where(restage, naddr, bstart)

            @pl.when(adv)
            def _():
                @pl.when(issue)
                def _():
                    @pl.when(restage)
                    def _():
                        pltpu.sync_copy(idx_hbm.at[pl.ds(naddr, BLK)], blk_v)

                    issue_gather(naddr - nbstart, lax.rem(c + R - 1, R))

                ws = lax.rem(c + 1, R)
                pltpu.make_async_copy(feat_hbm.at[pl.ds(0, G)],
                                      g_v.at[ws], gsem.at[ws]).wait()

            nm = jnp.where(hit, m + 1, m)
            seg_start = jnp.where(hit, seg_end, seg_start)
            seg_end = jnp.where(hit, rs_at(nm + 1), seg_end)
            accs = tuple(jnp.where(hit, zero, a) for a in accs)
            c = jnp.where(adv, c + 1, c)
            return (c, cur, nm, seg_start, seg_end, nbstart, accs)

        st0 = (jnp.int32(0), e0, jnp.int32(0), e0, rs_at(1),
               gstart, zeros_kc)
        lax.fori_loop(0, nevents, event, st0)

        for i in range(W):
            @pl.when(mcount > i)
            def _(i=i):
                ds = lax.rem(mcount - 1 - i, W)
                pltpu.make_async_copy(feat_hbm.at[0], row_v.at[ds, 0],
                                      wsem.at[ds]).wait()

    return pool(in_features, idx32, rs32)
